# baseline (speedup 1.0000x reference)
import functools

import jax
import jax.numpy as jnp
from jax import lax
from jax.experimental import pallas as pl
from jax.experimental.pallas import tpu as pltpu
from jax.experimental.pallas import tpu_sc as plsc

ENC = 8192
VOCAB = 100000
HID = 1024
ATT = 384

NC = 2
NS = 16
CHUNK = 3136
HALF = NS * CHUNK
VP = NC * HALF
TRASH = HALF
ACC = HALF + 16
EPT = ENC // NS
BLK = 8192


def _tc_body(bpg_ref, badd_ref, hs_ref, wpg_ref, attn_ref, wad_ref,
             src_ref, pg_ref, pgs_ref):
    z = jnp.sum(hs_ref[...] * wpg_ref[...]) + bpg_ref[0]
    p = jax.nn.sigmoid(z)
    a = jnp.sum(attn_ref[...] * wad_ref[...], axis=1, keepdims=True)
    src_ref[...] = (1.0 - p) * jnp.maximum(a + badd_ref[0], 0.0)

    @pl.when(pl.program_id(0) == 0)
    def _():
        pg_ref[...] = jnp.full((1, 128), p, jnp.float32)
        pgs_ref[0] = p


def _tc_call(bpg, badd, hs, wpg, attn, wad):
    return pl.pallas_call(
        _tc_body,
        grid=(ENC // BLK,),
        in_specs=[
            pl.BlockSpec(memory_space=pltpu.SMEM),
            pl.BlockSpec(memory_space=pltpu.SMEM),
            pl.BlockSpec((1, HID), lambda i: (0, 0)),
            pl.BlockSpec((1, HID), lambda i: (0, 0)),
            pl.BlockSpec((BLK, ATT), lambda i: (i, 0)),
            pl.BlockSpec((1, ATT), lambda i: (0, 0)),
        ],
        out_specs=[
            pl.BlockSpec((BLK, 1), lambda i: (i, 0)),
            pl.BlockSpec((1, 128), lambda i: (0, 0)),
            pl.BlockSpec(memory_space=pltpu.SMEM),
        ],
        out_shape=[
            jax.ShapeDtypeStruct((ENC, 1), jnp.float32),
            jax.ShapeDtypeStruct((1, 128), jnp.float32),
            jax.ShapeDtypeStruct((1,), jnp.float32),
        ],
    )(bpg, badd, hs, wpg, attn, wad)


_SC_MESH = plsc.VectorSubcoreMesh(core_axis_name="c", subcore_axis_name="s")


NW = NC * NS
TAIL_W = NW - 1
TAIL_G0 = TAIL_W * CHUNK
TAIL_N = VOCAB - TAIL_G0


@functools.partial(
    pl.kernel,
    out_type=jax.ShapeDtypeStruct((VOCAB,), jnp.float32),
    mesh=_SC_MESH,
    compiler_params=pltpu.CompilerParams(needs_layout_passes=False),
    scratch_types=[
        pltpu.VMEM((ENC,), jnp.int32),
        pltpu.VMEM((ENC,), jnp.float32),
        pltpu.VMEM((16,), jnp.float32),
        pltpu.VMEM((CHUNK,), jnp.float32),
        pltpu.SemaphoreType.DMA,
        pltpu.SemaphoreType.DMA,
        pltpu.SemaphoreType.DMA,
        pltpu.SemaphoreType.DMA,
    ],
)
def _sc_kernel(vocab_hbm, ids_hbm, src_hbm, pg_hbm, out_hbm,
               ids_v, src_v, pg_v, chunk_v,
               sem_ids, sem_src, sem_chunk, sem_pg):
    c = lax.axis_index("c")
    s = lax.axis_index("s")
    w = c * NS + s
    g0 = w * CHUNK
    is_tail = w == TAIL_W

    @pl.when(is_tail)
    def _():
        pltpu.async_copy(vocab_hbm.at[pl.ds(TAIL_G0, TAIL_N)],
                         chunk_v.at[pl.ds(0, TAIL_N)], sem_chunk)

    @pl.when(jnp.logical_not(is_tail))
    def _():
        pltpu.async_copy(vocab_hbm.at[pl.ds(g0, CHUNK)],
                         chunk_v, sem_chunk)

    pg_cp = pltpu.async_copy(pg_hbm.at[pl.ds(0, 16)], pg_v, sem_pg)
    ids_cp = pltpu.async_copy(ids_hbm, ids_v, sem_ids)
    src_cp = pltpu.async_copy(src_hbm, src_v, sem_src)

    @pl.when(is_tail)
    def _():
        pltpu.make_async_copy(vocab_hbm.at[pl.ds(TAIL_G0, TAIL_N)],
                              chunk_v.at[pl.ds(0, TAIL_N)], sem_chunk).wait()

    @pl.when(jnp.logical_not(is_tail))
    def _():
        pltpu.make_async_copy(vocab_hbm.at[pl.ds(g0, CHUNK)],
                              chunk_v, sem_chunk).wait()

    pg_cp.wait()
    pg = pg_v[...]

    def scale_body(i, carry):
        for u in range(4):
            sl = pl.ds((i * 4 + u) * 16, 16)
            chunk_v[sl] = chunk_v[sl] * pg
        return carry

    lax.fori_loop(0, CHUNK // 64, scale_body, 0)

    ids_cp.wait()
    src_cp.wait()

    def scatter_body(i, carry):
        for u in range(8):
            sl = pl.ds((i * 8 + u) * 16, 16)
            lv = ids_v[sl] - g0
            inb = (lv >= 0) & (lv < CHUNK)
            lv = jnp.where(inb, lv, 0)
            plsc.addupdate_scatter(chunk_v, [lv], src_v[sl], mask=inb)
        return carry

    lax.fori_loop(0, ENC // 128, scatter_body, 0)

    @pl.when(is_tail)
    def _():
        pltpu.sync_copy(chunk_v.at[pl.ds(0, TAIL_N)],
                        out_hbm.at[pl.ds(TAIL_G0, TAIL_N)])

    @pl.when(jnp.logical_not(is_tail))
    def _():
        pltpu.sync_copy(chunk_v, out_hbm.at[pl.ds(g0, CHUNK)])


def kernel(input_ids, attentions, hidden_states,
           output_vocabulary_probabilities, W_pgen, b_pgen, W_add, b_add):
    attn = attentions.reshape(ENC, ATT)
    hs = hidden_states.reshape(1, HID)
    wpg = W_pgen.reshape(1, HID)
    wad = W_add.reshape(1, ATT)
    vocab = output_vocabulary_probabilities.reshape(VOCAB)
    ids = input_ids.reshape(ENC).astype(jnp.int32)

    src2d, pgv, pgs = _tc_call(b_pgen, b_add, hs, wpg, attn, wad)
    out_p = _sc_kernel(vocab, ids, src2d.reshape(ENC), pgv.reshape(128))
    out = out_p.reshape(1, 1, VOCAB)
    p_gen = pgs.reshape(1, 1, 1)
    return (out, p_gen)


# speedup vs baseline: 1.0251x; 1.0251x over previous
import functools

import jax
import jax.numpy as jnp
from jax import lax
from jax.experimental import pallas as pl
from jax.experimental.pallas import tpu as pltpu
from jax.experimental.pallas import tpu_sc as plsc

ENC = 8192
VOCAB = 100000
HID = 1024
ATT = 384

NC = 2
NS = 16
CHUNK = 3136
HALF = NS * CHUNK
VP = NC * HALF
TRASH = HALF
ACC = HALF + 16
EPT = ENC // NS
BLK = 4096


def _tc_body(bpg_ref, badd_ref, hs_ref, wpg_ref, attn_ref, wad_ref,
             src_ref, pg_ref, pgs_ref):
    z = jnp.sum(hs_ref[...] * wpg_ref[...]) + bpg_ref[0]
    p = jax.nn.sigmoid(z)
    a = jnp.sum(attn_ref[...] * wad_ref[...], axis=1, keepdims=True)
    src_ref[...] = (1.0 - p) * jnp.maximum(a + badd_ref[0], 0.0)

    @pl.when(pl.program_id(0) == 0)
    def _():
        pg_ref[...] = jnp.full((1, 128), p, jnp.float32)
        pgs_ref[0] = p


def _tc_call(bpg, badd, hs, wpg, attn, wad):
    return pl.pallas_call(
        _tc_body,
        grid=(ENC // BLK,),
        in_specs=[
            pl.BlockSpec(memory_space=pltpu.SMEM),
            pl.BlockSpec(memory_space=pltpu.SMEM),
            pl.BlockSpec((1, HID), lambda i: (0, 0)),
            pl.BlockSpec((1, HID), lambda i: (0, 0)),
            pl.BlockSpec((BLK, ATT), lambda i: (i, 0)),
            pl.BlockSpec((1, ATT), lambda i: (0, 0)),
        ],
        out_specs=[
            pl.BlockSpec((BLK, 1), lambda i: (i, 0)),
            pl.BlockSpec((1, 128), lambda i: (0, 0)),
            pl.BlockSpec(memory_space=pltpu.SMEM),
        ],
        out_shape=[
            jax.ShapeDtypeStruct((ENC, 1), jnp.float32),
            jax.ShapeDtypeStruct((1, 128), jnp.float32),
            jax.ShapeDtypeStruct((1,), jnp.float32),
        ],
    )(bpg, badd, hs, wpg, attn, wad)


_SC_MESH = plsc.VectorSubcoreMesh(core_axis_name="c", subcore_axis_name="s")


NW = NC * NS
TAIL_W = NW - 1
TAIL_G0 = TAIL_W * CHUNK
TAIL_N = VOCAB - TAIL_G0


@functools.partial(
    pl.kernel,
    out_type=jax.ShapeDtypeStruct((VOCAB,), jnp.float32),
    mesh=_SC_MESH,
    compiler_params=pltpu.CompilerParams(
        needs_layout_passes=False,
        disable_bounds_checks=True,
        disable_semaphore_checks=True,
        skip_device_barrier=True,
    ),
    scratch_types=[
        pltpu.VMEM((ENC,), jnp.int32),
        pltpu.VMEM((ENC,), jnp.float32),
        pltpu.VMEM((16,), jnp.float32),
        pltpu.VMEM((CHUNK,), jnp.float32),
        pltpu.SemaphoreType.DMA,
        pltpu.SemaphoreType.DMA,
        pltpu.SemaphoreType.DMA,
        pltpu.SemaphoreType.DMA,
    ],
)
def _sc_kernel(vocab_hbm, ids_hbm, src_hbm, pg_hbm, out_hbm,
               ids_v, src_v, pg_v, chunk_v,
               sem_ids, sem_src, sem_chunk, sem_pg):
    c = lax.axis_index("c")
    s = lax.axis_index("s")
    w = c * NS + s
    g0 = w * CHUNK
    is_tail = w == TAIL_W

    @pl.when(is_tail)
    def _():
        pltpu.async_copy(vocab_hbm.at[pl.ds(TAIL_G0, TAIL_N)],
                         chunk_v.at[pl.ds(0, TAIL_N)], sem_chunk)

    @pl.when(jnp.logical_not(is_tail))
    def _():
        pltpu.async_copy(vocab_hbm.at[pl.ds(g0, CHUNK)],
                         chunk_v, sem_chunk)

    pg_cp = pltpu.async_copy(pg_hbm.at[pl.ds(0, 16)], pg_v, sem_pg)
    ids_cp = pltpu.async_copy(ids_hbm, ids_v, sem_ids)
    src_cp = pltpu.async_copy(src_hbm, src_v, sem_src)

    @pl.when(is_tail)
    def _():
        pltpu.make_async_copy(vocab_hbm.at[pl.ds(TAIL_G0, TAIL_N)],
                              chunk_v.at[pl.ds(0, TAIL_N)], sem_chunk).wait()

    @pl.when(jnp.logical_not(is_tail))
    def _():
        pltpu.make_async_copy(vocab_hbm.at[pl.ds(g0, CHUNK)],
                              chunk_v, sem_chunk).wait()

    pg_cp.wait()
    pg = pg_v[...]

    def scale_body(i, carry):
        for u in range(4):
            sl = pl.ds((i * 4 + u) * 16, 16)
            chunk_v[sl] = chunk_v[sl] * pg
        return carry

    lax.fori_loop(0, CHUNK // 64, scale_body, 0)

    ids_cp.wait()
    src_cp.wait()

    def scatter_body(i, carry):
        for u in range(8):
            sl = pl.ds((i * 8 + u) * 16, 16)
            lv = ids_v[sl] - g0
            inb = (lv >= 0) & (lv < CHUNK)
            lv = jnp.where(inb, lv, 0)
            plsc.addupdate_scatter(chunk_v, [lv], src_v[sl], mask=inb)
        return carry

    lax.fori_loop(0, ENC // 128, scatter_body, 0)

    @pl.when(is_tail)
    def _():
        pltpu.sync_copy(chunk_v.at[pl.ds(0, TAIL_N)],
                        out_hbm.at[pl.ds(TAIL_G0, TAIL_N)])

    @pl.when(jnp.logical_not(is_tail))
    def _():
        pltpu.sync_copy(chunk_v, out_hbm.at[pl.ds(g0, CHUNK)])


def kernel(input_ids, attentions, hidden_states,
           output_vocabulary_probabilities, W_pgen, b_pgen, W_add, b_add):
    attn = attentions.reshape(ENC, ATT)
    hs = hidden_states.reshape(1, HID)
    wpg = W_pgen.reshape(1, HID)
    wad = W_add.reshape(1, ATT)
    vocab = output_vocabulary_probabilities.reshape(VOCAB)
    ids = input_ids.reshape(ENC).astype(jnp.int32)

    src2d, pgv, pgs = _tc_call(b_pgen, b_add, hs, wpg, attn, wad)
    out_p = _sc_kernel(vocab, ids, src2d.reshape(ENC), pgv.reshape(128))
    out = out_p.reshape(1, 1, VOCAB)
    p_gen = pgs.reshape(1, 1, 1)
    return (out, p_gen)


# speedup vs baseline: 1.0265x; 1.0014x over previous
import functools

import jax
import jax.numpy as jnp
from jax import lax
from jax.experimental import pallas as pl
from jax.experimental.pallas import tpu as pltpu
from jax.experimental.pallas import tpu_sc as plsc

ENC = 8192
VOCAB = 100000
HID = 1024
ATT = 384

NC = 2
NS = 16
CHUNK = 3136
HALF = NS * CHUNK
VP = NC * HALF
TRASH = HALF
ACC = HALF + 16
EPT = ENC // NS
BLK = 4096


def _tc_body(bpg_ref, badd_ref, hs_ref, wpg_ref, attn_ref, wad_ref,
             src_ref, pg_ref, pgs_ref):
    z = jnp.sum(hs_ref[...] * wpg_ref[...]) + bpg_ref[0]
    p = jax.nn.sigmoid(z)
    a = jnp.sum(attn_ref[...] * wad_ref[...], axis=1, keepdims=True)
    src_ref[...] = (1.0 - p) * jnp.maximum(a + badd_ref[0], 0.0)

    @pl.when(pl.program_id(0) == 0)
    def _():
        pg_ref[...] = jnp.full((1, 128), p, jnp.float32)
        pgs_ref[0] = p


def _tc_call(bpg, badd, hs, wpg, attn, wad):
    return pl.pallas_call(
        _tc_body,
        grid=(ENC // BLK,),
        in_specs=[
            pl.BlockSpec(memory_space=pltpu.SMEM),
            pl.BlockSpec(memory_space=pltpu.SMEM),
            pl.BlockSpec((1, HID), lambda i: (0, 0)),
            pl.BlockSpec((1, HID), lambda i: (0, 0)),
            pl.BlockSpec((BLK, ATT), lambda i: (i, 0)),
            pl.BlockSpec((1, ATT), lambda i: (0, 0)),
        ],
        out_specs=[
            pl.BlockSpec((BLK, 1), lambda i: (i, 0)),
            pl.BlockSpec((1, 128), lambda i: (0, 0)),
            pl.BlockSpec(memory_space=pltpu.SMEM),
        ],
        out_shape=[
            jax.ShapeDtypeStruct((ENC, 1), jnp.float32),
            jax.ShapeDtypeStruct((1, 128), jnp.float32),
            jax.ShapeDtypeStruct((1,), jnp.float32),
        ],
    )(bpg, badd, hs, wpg, attn, wad)


_SC_MESH = plsc.VectorSubcoreMesh(core_axis_name="c", subcore_axis_name="s")


NW = NC * NS
TAIL_W = NW - 1
TAIL_G0 = TAIL_W * CHUNK
TAIL_N = VOCAB - TAIL_G0


@functools.partial(
    pl.kernel,
    out_type=jax.ShapeDtypeStruct((VOCAB,), jnp.float32),
    mesh=_SC_MESH,
    compiler_params=pltpu.CompilerParams(
        needs_layout_passes=False,
        disable_bounds_checks=True,
        disable_semaphore_checks=True,
        skip_device_barrier=True,
    ),
    scratch_types=[
        pltpu.VMEM((ENC,), jnp.int32),
        pltpu.VMEM((ENC,), jnp.float32),
        pltpu.VMEM((16,), jnp.float32),
        pltpu.VMEM((CHUNK,), jnp.float32),
        pltpu.SemaphoreType.DMA,
        pltpu.SemaphoreType.DMA,
        pltpu.SemaphoreType.DMA,
        pltpu.SemaphoreType.DMA,
    ],
)
def _sc_kernel(vocab_hbm, ids_hbm, src_hbm, pg_hbm, out_hbm,
               ids_v, src_v, pg_v, chunk_v,
               sem_ids, sem_src, sem_chunk, sem_pg):
    c = lax.axis_index("c")
    s = lax.axis_index("s")
    w = c * NS + s
    g0 = w * CHUNK
    is_tail = w == TAIL_W

    @pl.when(is_tail)
    def _():
        pltpu.async_copy(vocab_hbm.at[pl.ds(TAIL_G0, TAIL_N)],
                         chunk_v.at[pl.ds(0, TAIL_N)], sem_chunk)

    @pl.when(jnp.logical_not(is_tail))
    def _():
        pltpu.async_copy(vocab_hbm.at[pl.ds(g0, CHUNK)],
                         chunk_v, sem_chunk)

    pg_cp = pltpu.async_copy(pg_hbm.at[pl.ds(0, 16)], pg_v, sem_pg)
    ids_cp = pltpu.async_copy(ids_hbm, ids_v, sem_ids)
    src_cp = pltpu.async_copy(src_hbm, src_v, sem_src)

    @pl.when(is_tail)
    def _():
        pltpu.make_async_copy(vocab_hbm.at[pl.ds(TAIL_G0, TAIL_N)],
                              chunk_v.at[pl.ds(0, TAIL_N)], sem_chunk).wait()

    @pl.when(jnp.logical_not(is_tail))
    def _():
        pltpu.make_async_copy(vocab_hbm.at[pl.ds(g0, CHUNK)],
                              chunk_v, sem_chunk).wait()

    pg_cp.wait()
    pg = pg_v[...]
    _ns = jax.named_scope

    def scale_body(i, carry):
        for u in range(4):
            sl = pl.ds((i * 4 + u) * 16, 16)
            chunk_v[sl] = chunk_v[sl] * pg
        return carry

    with _ns("scale"):
        lax.fori_loop(0, CHUNK // 64, scale_body, 0)

    with _ns("wait_ids_src"):
        ids_cp.wait()
        src_cp.wait()

    def scatter_body(i, carry):
        for u in range(8):
            sl = pl.ds((i * 8 + u) * 16, 16)
            lv = ids_v[sl] - g0
            inb = (lv >= 0) & (lv < CHUNK)
            lv = jnp.where(inb, lv, 0)
            plsc.addupdate_scatter(chunk_v, [lv], src_v[sl], mask=inb)
        return carry

    with _ns("scatter"):
        lax.fori_loop(0, ENC // 128, scatter_body, 0)

    @pl.when(is_tail)
    def _():
        pltpu.sync_copy(chunk_v.at[pl.ds(0, TAIL_N)],
                        out_hbm.at[pl.ds(TAIL_G0, TAIL_N)])

    @pl.when(jnp.logical_not(is_tail))
    def _():
        pltpu.sync_copy(chunk_v, out_hbm.at[pl.ds(g0, CHUNK)])


def kernel(input_ids, attentions, hidden_states,
           output_vocabulary_probabilities, W_pgen, b_pgen, W_add, b_add):
    attn = attentions.reshape(ENC, ATT)
    hs = hidden_states.reshape(1, HID)
    wpg = W_pgen.reshape(1, HID)
    wad = W_add.reshape(1, ATT)
    vocab = output_vocabulary_probabilities.reshape(VOCAB)
    ids = input_ids.reshape(ENC).astype(jnp.int32)

    src2d, pgv, pgs = _tc_call(b_pgen, b_add, hs, wpg, attn, wad)
    out_p = _sc_kernel(vocab, ids, src2d.reshape(ENC), pgv.reshape(128))
    out = out_p.reshape(1, 1, VOCAB)
    p_gen = pgs.reshape(1, 1, 1)
    return (out, p_gen)


# speedup vs baseline: 1.2861x; 1.2528x over previous
import functools

import jax
import jax.numpy as jnp
from jax import lax
from jax.experimental import pallas as pl
from jax.experimental.pallas import tpu as pltpu
from jax.experimental.pallas import tpu_sc as plsc

ENC = 8192
VOCAB = 100000
HID = 1024
ATT = 384

NC = 2
NS = 16
CHUNK = 3200
BLK = 4096


def _tc_body(bpg_ref, badd_ref, hs_ref, wpg_ref, attn_ref, wad_ref,
             src_ref, pg_ref, pgs_ref):
    z = jnp.sum(hs_ref[...] * wpg_ref[...]) + bpg_ref[0]
    p = jax.nn.sigmoid(z)
    a = jnp.sum(attn_ref[...] * wad_ref[...], axis=2)
    src_ref[...] = (1.0 - p) * jnp.maximum(a + badd_ref[0], 0.0)

    @pl.when(pl.program_id(0) == 0)
    def _():
        pg_ref[...] = jnp.full((1, 128), p, jnp.float32)
        pgs_ref[0] = p


def _tc_call(bpg, badd, hs, wpg, attn, wad):
    return pl.pallas_call(
        _tc_body,
        grid=(ENC // BLK,),
        in_specs=[
            pl.BlockSpec(memory_space=pltpu.SMEM),
            pl.BlockSpec(memory_space=pltpu.SMEM),
            pl.BlockSpec((1, HID), lambda i: (0, 0)),
            pl.BlockSpec((1, HID), lambda i: (0, 0)),
            pl.BlockSpec((BLK // 128, 128, ATT), lambda i: (i, 0, 0)),
            pl.BlockSpec((1, 1, ATT), lambda i: (0, 0, 0)),
        ],
        out_specs=[
            pl.BlockSpec((BLK // 128, 128), lambda i: (i, 0)),
            pl.BlockSpec((1, 128), lambda i: (0, 0)),
            pl.BlockSpec(memory_space=pltpu.SMEM),
        ],
        out_shape=[
            jax.ShapeDtypeStruct((ENC // 128, 128), jnp.float32),
            jax.ShapeDtypeStruct((1, 128), jnp.float32),
            jax.ShapeDtypeStruct((1,), jnp.float32),
        ],
    )(bpg, badd, hs, wpg, attn, wad)


_SC_MESH = plsc.VectorSubcoreMesh(core_axis_name="c", subcore_axis_name="s")


NW = NC * NS
TAIL_W = NW - 1
TAIL_G0 = TAIL_W * CHUNK
TAIL_N = VOCAB - TAIL_G0


@functools.partial(
    pl.kernel,
    out_type=jax.ShapeDtypeStruct((1, 1, VOCAB), jnp.float32),
    mesh=_SC_MESH,
    compiler_params=pltpu.CompilerParams(
        needs_layout_passes=False,
        disable_bounds_checks=True,
        disable_semaphore_checks=True,
        skip_device_barrier=True,
    ),
    scratch_types=[
        pltpu.VMEM((ENC,), jnp.int32),
        pltpu.VMEM((ENC,), jnp.float32),
        pltpu.VMEM((16,), jnp.float32),
        pltpu.VMEM((CHUNK,), jnp.float32),
        pltpu.SemaphoreType.DMA,
        pltpu.SemaphoreType.DMA,
        pltpu.SemaphoreType.DMA,
        pltpu.SemaphoreType.DMA,
    ],
)
def _sc_kernel(vocab_hbm, ids_hbm, src_hbm, pg_hbm, out_hbm,
               ids_v, src_v, pg_v, chunk_v,
               sem_ids, sem_src, sem_chunk, sem_pg):
    c = lax.axis_index("c")
    s = lax.axis_index("s")
    w = c * NS + s
    g0 = w * CHUNK
    is_tail = w == TAIL_W

    @pl.when(is_tail)
    def _():
        pltpu.async_copy(vocab_hbm.at[0, 0, pl.ds(TAIL_G0, TAIL_N)],
                         chunk_v.at[pl.ds(0, TAIL_N)], sem_chunk)

    @pl.when(jnp.logical_not(is_tail))
    def _():
        pltpu.async_copy(vocab_hbm.at[0, 0, pl.ds(g0, CHUNK)],
                         chunk_v, sem_chunk)

    pg_cp = pltpu.async_copy(pg_hbm.at[pl.ds(0, 16)], pg_v, sem_pg)
    ids_cp = pltpu.async_copy(ids_hbm, ids_v, sem_ids)
    src_cp = pltpu.async_copy(src_hbm, src_v, sem_src)

    @pl.when(is_tail)
    def _():
        pltpu.make_async_copy(vocab_hbm.at[0, 0, pl.ds(TAIL_G0, TAIL_N)],
                              chunk_v.at[pl.ds(0, TAIL_N)], sem_chunk).wait()

    @pl.when(jnp.logical_not(is_tail))
    def _():
        pltpu.make_async_copy(vocab_hbm.at[0, 0, pl.ds(g0, CHUNK)],
                              chunk_v, sem_chunk).wait()

    pg_cp.wait()
    pg = pg_v[...]
    _ns = jax.named_scope

    def scale_body(i, carry):
        for u in range(4):
            sl = pl.ds((i * 4 + u) * 16, 16)
            chunk_v[sl] = chunk_v[sl] * pg
        return carry

    with _ns("scale"):
        lax.fori_loop(0, CHUNK // 64, scale_body, 0)

    with _ns("wait_ids_src"):
        ids_cp.wait()
        src_cp.wait()

    def scatter_body(i, carry):
        for u in range(8):
            sl = pl.ds((i * 8 + u) * 16, 16)
            lv = ids_v[sl] - g0
            inb = (lv >= 0) & (lv < CHUNK)
            lv = jnp.where(inb, lv, 0)
            plsc.addupdate_scatter(chunk_v, [lv], src_v[sl], mask=inb)
        return carry

    with _ns("scatter"):
        lax.fori_loop(0, ENC // 128, scatter_body, 0)

    @pl.when(is_tail)
    def _():
        pltpu.sync_copy(chunk_v.at[pl.ds(0, TAIL_N)],
                        out_hbm.at[0, 0, pl.ds(TAIL_G0, TAIL_N)])

    @pl.when(jnp.logical_not(is_tail))
    def _():
        pltpu.sync_copy(chunk_v, out_hbm.at[0, 0, pl.ds(g0, CHUNK)])


def kernel(input_ids, attentions, hidden_states,
           output_vocabulary_probabilities, W_pgen, b_pgen, W_add, b_add):
    attn = attentions.reshape(ENC // 128, 128, ATT)
    hs = hidden_states.reshape(1, HID)
    wpg = W_pgen.reshape(1, HID)
    wad = W_add.reshape(1, 1, ATT)
    ids = input_ids.reshape(ENC).astype(jnp.int32)

    src2d, pgv, pgs = _tc_call(b_pgen, b_add, hs, wpg, attn, wad)
    out = _sc_kernel(output_vocabulary_probabilities, ids,
                     src2d.reshape(ENC), pgv.reshape(128))
    p_gen = pgs.reshape(1, 1, 1)
    return (out, p_gen)


# speedup vs baseline: 1.2964x; 1.0081x over previous
import functools

import jax
import jax.numpy as jnp
from jax import lax
from jax.experimental import pallas as pl
from jax.experimental.pallas import tpu as pltpu
from jax.experimental.pallas import tpu_sc as plsc

ENC = 8192
VOCAB = 100000
HID = 1024
ATT = 384

NC = 2
NS = 16
CHUNK = 3200
BLK = 4096


def _tc_body(bpg_ref, badd_ref, hs_ref, wpg_ref, attn_ref, wad_ref,
             src_ref, pg_ref, pgs_ref):
    z = jnp.sum(hs_ref[...] * wpg_ref[...]) + bpg_ref[0]
    p = jax.nn.sigmoid(z)
    a = jnp.sum(attn_ref[...] * wad_ref[...], axis=2)
    src_ref[...] = (1.0 - p) * jnp.maximum(a + badd_ref[0], 0.0)

    @pl.when(pl.program_id(0) == 0)
    def _():
        pg_ref[...] = jnp.full((1, 128), p, jnp.float32)
        pgs_ref[0] = p


def _tc_call(bpg, badd, hs, wpg, attn, wad):
    return pl.pallas_call(
        _tc_body,
        grid=(ENC // BLK,),
        in_specs=[
            pl.BlockSpec(memory_space=pltpu.SMEM),
            pl.BlockSpec(memory_space=pltpu.SMEM),
            pl.BlockSpec((1, HID), lambda i: (0, 0)),
            pl.BlockSpec((1, HID), lambda i: (0, 0)),
            pl.BlockSpec((BLK // 128, 128, ATT), lambda i: (i, 0, 0)),
            pl.BlockSpec((1, 1, ATT), lambda i: (0, 0, 0)),
        ],
        out_specs=[
            pl.BlockSpec((BLK // 128, 128), lambda i: (i, 0)),
            pl.BlockSpec((1, 128), lambda i: (0, 0)),
            pl.BlockSpec(memory_space=pltpu.SMEM),
        ],
        out_shape=[
            jax.ShapeDtypeStruct((ENC // 128, 128), jnp.float32),
            jax.ShapeDtypeStruct((1, 128), jnp.float32),
            jax.ShapeDtypeStruct((1,), jnp.float32),
        ],
    )(bpg, badd, hs, wpg, attn, wad)


_SC_MESH = plsc.VectorSubcoreMesh(core_axis_name="c", subcore_axis_name="s")


NW = NC * NS
TAIL_W = NW - 1
TAIL_G0 = TAIL_W * CHUNK
TAIL_N = VOCAB - TAIL_G0


@functools.partial(
    pl.kernel,
    out_type=jax.ShapeDtypeStruct((1, 1, VOCAB), jnp.float32),
    mesh=_SC_MESH,
    compiler_params=pltpu.CompilerParams(
        needs_layout_passes=False,
        disable_bounds_checks=True,
        disable_semaphore_checks=True,
        skip_device_barrier=True,
    ),
    scratch_types=[
        pltpu.VMEM((ENC,), jnp.int32),
        pltpu.VMEM((ENC,), jnp.float32),
        pltpu.VMEM((16,), jnp.float32),
        pltpu.VMEM((CHUNK,), jnp.float32),
        pltpu.VMEM((ENC,), jnp.int32),
        pltpu.SemaphoreType.DMA,
        pltpu.SemaphoreType.DMA,
        pltpu.SemaphoreType.DMA,
        pltpu.SemaphoreType.DMA,
    ],
)
def _sc_kernel(vocab_hbm, ids_hbm, src_hbm, pg_hbm, out_hbm,
               ids_v, src_v, pg_v, chunk_v, work_v,
               sem_ids, sem_src, sem_chunk, sem_pg):
    c = lax.axis_index("c")
    s = lax.axis_index("s")
    w = c * NS + s
    g0 = w * CHUNK
    is_tail = w == TAIL_W

    @pl.when(is_tail)
    def _():
        pltpu.async_copy(vocab_hbm.at[0, 0, pl.ds(TAIL_G0, TAIL_N)],
                         chunk_v.at[pl.ds(0, TAIL_N)], sem_chunk)

    @pl.when(jnp.logical_not(is_tail))
    def _():
        pltpu.async_copy(vocab_hbm.at[0, 0, pl.ds(g0, CHUNK)],
                         chunk_v, sem_chunk)

    pg_cp = pltpu.async_copy(pg_hbm.at[pl.ds(0, 16)], pg_v, sem_pg)
    ids_cp = pltpu.async_copy(ids_hbm, ids_v, sem_ids)
    src_cp = pltpu.async_copy(src_hbm, src_v, sem_src)

    @pl.when(is_tail)
    def _():
        pltpu.make_async_copy(vocab_hbm.at[0, 0, pl.ds(TAIL_G0, TAIL_N)],
                              chunk_v.at[pl.ds(0, TAIL_N)], sem_chunk).wait()

    @pl.when(jnp.logical_not(is_tail))
    def _():
        pltpu.make_async_copy(vocab_hbm.at[0, 0, pl.ds(g0, CHUNK)],
                              chunk_v, sem_chunk).wait()

    pg_cp.wait()
    pg = pg_v[...]
    _ns = jax.named_scope

    def scale_body(i, carry):
        for u in range(4):
            sl = pl.ds((i * 4 + u) * 16, 16)
            chunk_v[sl] = chunk_v[sl] * pg
        return carry

    with _ns("scale"):
        lax.fori_loop(0, CHUNK // 64, scale_body, 0)

    with _ns("wait_ids_src"):
        ids_cp.wait()
        src_cp.wait()

    def scatter_body(i, carry):
        for u in range(8):
            sl = pl.ds((i * 8 + u) * 16, 16)
            lv = ids_v[sl] - g0
            inb = (lv >= 0) & (lv < CHUNK)
            lv = jnp.where(inb, lv, 0)
            plsc.addupdate_scatter(chunk_v, [lv], src_v[sl], mask=inb)
        return carry

    with _ns("scatter"):
        lax.fori_loop(0, ENC // 128, scatter_body, 0)

    @pl.when(is_tail)
    def _():
        pltpu.sync_copy(chunk_v.at[pl.ds(0, TAIL_N)],
                        out_hbm.at[0, 0, pl.ds(TAIL_G0, TAIL_N)])

    @pl.when(jnp.logical_not(is_tail))
    def _():
        pltpu.sync_copy(chunk_v, out_hbm.at[0, 0, pl.ds(g0, CHUNK)])


def kernel(input_ids, attentions, hidden_states,
           output_vocabulary_probabilities, W_pgen, b_pgen, W_add, b_add):
    attn = attentions.reshape(ENC // 128, 128, ATT)
    hs = hidden_states.reshape(1, HID)
    wpg = W_pgen.reshape(1, HID)
    wad = W_add.reshape(1, 1, ATT)
    ids = input_ids.reshape(ENC).astype(jnp.int32)

    src2d, pgv, pgs = _tc_call(b_pgen, b_add, hs, wpg, attn, wad)
    out = _sc_kernel(output_vocabulary_probabilities, ids,
                     src2d.reshape(ENC), pgv.reshape(128))
    p_gen = pgs.reshape(1, 1, 1)
    return (out, p_gen)
